# Initial kernel scaffold; baseline (speedup 1.0000x reference)
import jax
import jax.numpy as jnp
from jax.experimental import pallas as pl


def kernel(x, pos_table):
    raise NotImplementedError("write your pallas kernel here")


# speedup vs baseline: 1.4829x; 1.4829x over previous
import jax
import jax.numpy as jnp
from jax.experimental import pallas as pl


def _add_body(p_ref, x_ref, o_ref):
    o_ref[...] = x_ref[...] + p_ref[...]


def kernel(x, pos_table):
    B, S, D = x.shape
    BLK = 512
    grid = (S // BLK, B)
    return pl.pallas_call(
        _add_body,
        grid=grid,
        in_specs=[
            pl.BlockSpec((BLK, D), lambda s, b: (s, 0)),
            pl.BlockSpec((1, BLK, D), lambda s, b: (b, s, 0)),
        ],
        out_specs=pl.BlockSpec((1, BLK, D), lambda s, b: (b, s, 0)),
        out_shape=jax.ShapeDtypeStruct((B, S, D), x.dtype),
    )(pos_table, x)


# speedup vs baseline: 1.7264x; 1.1642x over previous
import jax
import jax.numpy as jnp
from jax.experimental import pallas as pl


def _add_body(p_ref, x_ref, o_ref):
    o_ref[...] = x_ref[...] + p_ref[...][None]


def kernel(x, pos_table):
    B, S, D = x.shape
    BLK = 512
    grid = (S // BLK,)
    return pl.pallas_call(
        _add_body,
        grid=grid,
        in_specs=[
            pl.BlockSpec((BLK, D), lambda s: (s, 0)),
            pl.BlockSpec((B, BLK, D), lambda s: (0, s, 0)),
        ],
        out_specs=pl.BlockSpec((B, BLK, D), lambda s: (0, s, 0)),
        out_shape=jax.ShapeDtypeStruct((B, S, D), x.dtype),
    )(pos_table, x)
